# baseline (speedup 1.0000x reference)
import jax
import jax.numpy as jnp
from jax.experimental import pallas as pl

B, L, K, H, T = 4, 2048, 48, 128, 64
BL = 1024


_C = 0.7071067811865476


def _gelu_pre(y):
    return y + y * jax.lax.erf(y)


def _ln(x, g, b, eps=1e-5):
    m = jnp.mean(x, axis=-1, keepdims=True)
    v = jnp.mean((x - m) ** 2, axis=-1, keepdims=True)
    return (x - m) * jax.lax.rsqrt(v + eps) * g + b


def _fused_kernel(h_V_ref, h_E_ref, t_emb_ref, mask_ref,
                   W1a_ref, W1b_ref, W1t_ref, b1_ref,
                   W2h_ref, b2_ref, W3s_ref, b3s_ref,
                   g1_ref, be1_ref, g2_ref, be2_ref,
                   Win_ref, bin_ref, Wouth_ref, bout_ref,
                   out_ref):
    b = pl.program_id(0)
    hv = h_V_ref[0]
    te = t_emb_ref[pl.ds(b, 1), :]

    node_pre = (
        jnp.dot(hv, W1a_ref[...], preferred_element_type=jnp.float32)
        + jnp.dot(te, W1t_ref[...], preferred_element_type=jnp.float32)
        + b1_ref[...]
    )

    he = h_E_ref[0].reshape(BL * K, H)
    m = jnp.dot(he, W1b_ref[...], preferred_element_type=jnp.float32)
    np_b = node_pre.astype(jnp.bfloat16)
    m = m.reshape(BL, K, H).astype(jnp.bfloat16) + np_b[:, None, :]
    m = _gelu_pre(m).reshape(BL * K, H)
    m = jnp.dot(m, W2h_ref[...].astype(jnp.bfloat16),
                preferred_element_type=jnp.float32)
    m = _gelu_pre(m.astype(jnp.bfloat16) + b2_ref[...].astype(jnp.bfloat16))

    m3 = m.reshape(BL, K, H)
    m4 = (m3[:, :16, :] + m3[:, 16:32, :]) + m3[:, 32:, :]
    s = jnp.sum(m4.astype(jnp.float32), axis=1)
    dh = jnp.dot(s, W3s_ref[...], preferred_element_type=jnp.float32) + b3s_ref[...]

    hv1 = _ln(hv + dh, g1_ref[...], be1_ref[...])

    ff = _gelu_pre((jnp.dot(hv1, Win_ref[...], preferred_element_type=jnp.float32)
                    + bin_ref[...]).astype(jnp.bfloat16))
    dh2 = jnp.dot(ff, Wouth_ref[...].astype(jnp.bfloat16),
                  preferred_element_type=jnp.float32) + bout_ref[...]
    out = _ln(hv1 + dh2, g2_ref[...], be2_ref[...])
    out_ref[0] = out * mask_ref[0, :, :]


def kernel(h_V, h_E, E_idx, t_emb, mask_V, Wt, bt, W1, b1, W2, b2, W3, b3,
           g1, be1, g2, be2, Win, bin, Wout, bout):
    del E_idx
    c = _C
    W1a0 = W1[:H]
    W1a = c * W1a0
    W1b = c * W1[H:2 * H]
    W1t = c * (Wt @ W1a0 + W1[2 * H:])
    b1f = c * (bt @ W1a0 + b1)
    W2h = 0.5 * W2
    b2 = c * b2
    W3s = (1.0 / (2.0 * c * 30.0)) * W3
    b3s = (K / 30.0) * b3
    Win = c * Win
    bin = c * bin
    Wouth = (1.0 / (2.0 * c)) * Wout
    row = lambda x: x.reshape(1, -1)

    grid = (B, L // BL)
    full = lambda shape: pl.BlockSpec(shape, lambda b, l: (0,) * len(shape))

    out = pl.pallas_call(
        _fused_kernel,
        grid=grid,
        in_specs=[
            pl.BlockSpec((1, BL, H), lambda b, l: (b, l, 0)),
            pl.BlockSpec((1, BL, K, H), lambda b, l: (b, l, 0, 0)),
            pl.BlockSpec((B, T), lambda b, l: (0, 0)),
            pl.BlockSpec((1, BL, 1), lambda b, l: (b, l, 0)),
            full((H, H)), full((H, H)), full((T, H)), full((1, H)),
            full((H, H)), full((1, H)),
            full((H, H)), full((1, H)),
            full((1, H)), full((1, H)), full((1, H)), full((1, H)),
            full((H, 4 * H)), full((1, 4 * H)),
            full((4 * H, H)), full((1, H)),
        ],
        out_specs=pl.BlockSpec((1, BL, H), lambda b, l: (b, l, 0)),
        out_shape=jax.ShapeDtypeStruct((B, L, H), jnp.float32),
    )(h_V, h_E, t_emb, mask_V.reshape(B, L, 1),
      W1a, W1b, W1t, row(b1f),
      W2h, row(b2), W3s, row(b3s),
      row(g1), row(be1), row(g2), row(be2),
      Win, row(bin), Wouth, row(bout))
    return out


# speedup vs baseline: 1.0107x; 1.0107x over previous
import jax
import jax.numpy as jnp
from jax.experimental import pallas as pl

B, L, K, H, T = 4, 2048, 48, 128, 64
BL = 1024


_C = 0.7071067811865476


def _gelu_pre(y):
    return y + y * jax.lax.erf(y)


def _ln(x, g, b, eps=1e-5):
    m = jnp.mean(x, axis=-1, keepdims=True)
    v = jnp.mean((x - m) ** 2, axis=-1, keepdims=True)
    return (x - m) * jax.lax.rsqrt(v + eps) * g + b


def _fused_kernel(h_V_ref, h_E_ref, t_emb_ref, mask_ref,
                   W1a_ref, W1b_ref, W1t_ref, b1_ref,
                   W2h_ref, b2_ref, W3s_ref, b3s_ref,
                   g1_ref, be1_ref, g2_ref, be2_ref,
                   Win_ref, bin_ref, Wouth_ref, bout_ref,
                   out_ref):
    b = pl.program_id(0)
    hv = h_V_ref[0]
    te = t_emb_ref[pl.ds(b, 1), :]

    node_pre = (
        jnp.dot(hv, W1a_ref[...], preferred_element_type=jnp.float32)
        + jnp.dot(te, W1t_ref[...], preferred_element_type=jnp.float32)
        + b1_ref[...]
    )

    he = h_E_ref[0].reshape(BL * K, H)
    m = jnp.dot(he, W1b_ref[...], preferred_element_type=jnp.float32)
    np_b = node_pre.astype(jnp.bfloat16)
    m = m.reshape(BL, K, H).astype(jnp.bfloat16) + np_b[:, None, :]
    m = _gelu_pre(m).reshape(BL * K, H)
    m = jnp.dot(m, W2h_ref[...].astype(jnp.bfloat16),
                preferred_element_type=jnp.float32)
    m = _gelu_pre(m.astype(jnp.bfloat16) + b2_ref[...].astype(jnp.bfloat16))

    m3 = m.reshape(BL, K, H)
    m4 = (m3[:, :16, :] + m3[:, 16:32, :]) + m3[:, 32:, :]
    s = jnp.sum(m4.astype(jnp.float32), axis=1)
    dh = jnp.dot(s, W3s_ref[...], preferred_element_type=jnp.float32) + b3s_ref[...]

    hv1 = _ln(hv + dh, g1_ref[...], be1_ref[...])

    ff = _gelu_pre(jnp.dot(hv1, Win_ref[...], preferred_element_type=jnp.float32)
                   + bin_ref[...])
    dh2 = jnp.dot(ff, Wouth_ref[...], preferred_element_type=jnp.float32) + bout_ref[...]
    out = _ln(hv1 + dh2, g2_ref[...], be2_ref[...])
    out_ref[0] = out * mask_ref[0, :, :]


def kernel(h_V, h_E, E_idx, t_emb, mask_V, Wt, bt, W1, b1, W2, b2, W3, b3,
           g1, be1, g2, be2, Win, bin, Wout, bout):
    del E_idx
    c = _C
    W1a0 = W1[:H]
    W1a = c * W1a0
    W1b = c * W1[H:2 * H]
    W1t = c * (Wt @ W1a0 + W1[2 * H:])
    b1f = c * (bt @ W1a0 + b1)
    W2h = 0.5 * W2
    b2 = c * b2
    W3s = (1.0 / (2.0 * c * 30.0)) * W3
    b3s = (K / 30.0) * b3
    Win = c * Win
    bin = c * bin
    Wouth = (1.0 / (2.0 * c)) * Wout
    row = lambda x: x.reshape(1, -1)

    grid = (B, L // BL)
    full = lambda shape: pl.BlockSpec(shape, lambda b, l: (0,) * len(shape))

    out = pl.pallas_call(
        _fused_kernel,
        grid=grid,
        in_specs=[
            pl.BlockSpec((1, BL, H), lambda b, l: (b, l, 0)),
            pl.BlockSpec((1, BL, K, H), lambda b, l: (b, l, 0, 0)),
            pl.BlockSpec((B, T), lambda b, l: (0, 0)),
            pl.BlockSpec((1, BL, 1), lambda b, l: (b, l, 0)),
            full((H, H)), full((H, H)), full((T, H)), full((1, H)),
            full((H, H)), full((1, H)),
            full((H, H)), full((1, H)),
            full((1, H)), full((1, H)), full((1, H)), full((1, H)),
            full((H, 4 * H)), full((1, 4 * H)),
            full((4 * H, H)), full((1, H)),
        ],
        out_specs=pl.BlockSpec((1, BL, H), lambda b, l: (b, l, 0)),
        out_shape=jax.ShapeDtypeStruct((B, L, H), jnp.float32),
    )(h_V, h_E, t_emb, mask_V.reshape(B, L, 1),
      W1a, W1b, W1t, row(b1f),
      W2h, row(b2), W3s, row(b3s),
      row(g1), row(be1), row(g2), row(be2),
      Win, row(bin), Wouth, row(bout))
    return out
